# baseline (speedup 1.0000x reference)
import functools

import jax
import jax.numpy as jnp
from jax import lax
from jax.experimental import pallas as pl
from jax.experimental.pallas import tpu as pltpu
from jax.experimental.pallas import tpu_sc as plsc

N = 100000
DEG = 64
C = 3
V = 68
NG = DEG + 1
GP = 72

NC = 2
NS = 16

CHUNK_A = N // 10
BLK_A = 80
NBLK_A = CHUNK_A // BLK_A
GRP_PER_TEC = 13

_SC_PARAMS = pltpu.CompilerParams(needs_layout_passes=False)


def _lane_perms():
    iota = lax.iota(jnp.int32, 16)
    return [jnp.bitwise_and(iota + s, 15) for s in (8, 4, 2, 1)]


def _tree_total(v, perms):
    for p in perms:
        v = v + v.at[p].get(mode="promise_in_bounds")
    return v


def _node_neighbor_total(plane, adjfb, fb, perms):
    gs = [plsc.load_gather(plane, [adjfb[pl.ds(fb + 16 * u, 16)]])
          for u in range(4)]
    return _tree_total((gs[0] + gs[1]) + (gs[2] + gs[3]), perms)


CBLK = 2000
NCBLK = CHUNK_A // CBLK


def _sc_tanh(z):
    return 1.0 - 2.0 / (jnp.exp(z + z) + 1.0)


def _sel3(t, a0, a1, a2):
    return jnp.where(t == 0, a0, jnp.where(t == 1, a1, a2))


def _sc_layer0_body(xT, adjf, wv, x1out, pooledout, plane, adjb0, adjb1,
                    poolchunk, xk1b, xk2b, pk1b, pk2b, outb, wvb,
                    isem0, isem1):
    cid = lax.axis_index("c")
    sid = lax.axis_index("s")
    wid = sid * NC + cid
    comp = wid // 10
    chunk = wid % 10
    active = wid < 30

    @pl.when(active)
    def _():
        pltpu.sync_copy(xT.at[pl.ds(comp * N, N)], plane)
        pltpu.sync_copy(wv, wvb)
        lane15 = lax.iota(jnp.int32, 16) == 15
        perms = _lane_perms()
        base = chunk * CHUNK_A

        def in_slice(blk):
            return adjf.at[pl.ds((base + blk * BLK_A) * DEG, BLK_A * DEG)]

        bufs = ((adjb0, isem0), (adjb1, isem1))
        pltpu.async_copy(in_slice(0), adjb0, isem0)

        def round_fn(kb, _):
            for b, (adjb, isem) in enumerate(bufs):
                blk = kb * 2 + b

                @pl.when(blk < NBLK_A)
                def _():
                    @pl.when(blk + 1 < NBLK_A)
                    def _():
                        nxt = bufs[1 - b]
                        pltpu.async_copy(in_slice(blk + 1), nxt[0], nxt[1])

                    pltpu.make_async_copy(in_slice(blk), adjb, isem).wait()

                    def node_fn(v, _):
                        sv = _node_neighbor_total(plane, adjb, v * DEG, perms)
                        plsc.store_scatter(
                            poolchunk,
                            [jnp.full((16,), blk * BLK_A + v, jnp.int32)],
                            sv, mask=lane15)
                        return 0

                    lax.fori_loop(0, BLK_A, node_fn, 0, unroll=16)
            return 0

        lax.fori_loop(0, (NBLK_A + 1) // 2, round_fn, 0)
        pltpu.sync_copy(poolchunk, pooledout.at[pl.ds(comp * N + base,
                                                      CHUNK_A)])

    plsc.subcore_barrier()

    @pl.when(active)
    def _():
        k1 = lax.rem(comp + 1, 3)
        k2 = lax.rem(comp + 2, 3)
        base = chunk * CHUNK_A

        wA = wvb[pl.ds(0, 16)]
        wB = wvb[pl.ds(16, 16)]
        wC = wvb[pl.ds(24, 16)]

        def ext(i):
            if i < 16:
                return wA[i]
            if i < 32:
                return wB[i - 16]
            return wC[i - 24]

        def ws_(c, k):
            return ext(3 * c + k)

        def wc_(c, k, m):
            return ext(12 + 6 * c + 2 * k + m)

        hw = [[_sel3(comp, ws_(c, 0), ws_(c, 1), ws_(c, 2)),
               _sel3(k1, ws_(c, 0), ws_(c, 1), ws_(c, 2)),
               _sel3(k2, ws_(c, 0), ws_(c, 1), ws_(c, 2))]
              for c in range(3)]
        bsv = [ext(9 + c) for c in range(3)]
        cw = [_sel3(comp, wc_(0, k, 0), wc_(1, k, 0), wc_(2, k, 0))
              for k in range(3)]
        cpo = _sel3(comp, wc_(0, 0, 1), wc_(1, 1, 1), wc_(2, 2, 1)) * (1.0 / DEG)
        cpa = _sel3(comp, wc_(0, 1, 1), wc_(1, 2, 1), wc_(2, 0, 1)) * (1.0 / DEG)
        cpb = _sel3(comp, wc_(0, 2, 1), wc_(1, 0, 1), wc_(2, 1, 1)) * (1.0 / DEG)
        bcv = _sel3(comp, ext(30), ext(31), ext(32))

        def cblk_fn(cb, _):
            off = cb * CBLK
            cps = [
                pltpu.async_copy(xT.at[pl.ds(k1 * N + base + off, CBLK)],
                                 xk1b, isem0),
                pltpu.async_copy(xT.at[pl.ds(k2 * N + base + off, CBLK)],
                                 xk2b, isem0),
                pltpu.async_copy(
                    pooledout.at[pl.ds(k1 * N + base + off, CBLK)],
                    pk1b, isem0),
                pltpu.async_copy(
                    pooledout.at[pl.ds(k2 * N + base + off, CBLK)],
                    pk2b, isem0),
            ]
            for cp in cps:
                cp.wait()

            def vec_fn(v, _):
                o = v * 16
                xo = plane[pl.ds(base + off + o, 16)]
                xa = xk1b[pl.ds(o, 16)]
                xb = xk2b[pl.ds(o, 16)]
                po = poolchunk[pl.ds(off + o, 16)]
                pa = pk1b[pl.ds(o, 16)]
                pb = pk2b[pl.ds(o, 16)]
                h = [_sc_tanh(hw[c][0] * xo + hw[c][1] * xa + hw[c][2] * xb
                              + bsv[c]) for c in range(3)]
                z = (cw[0] * h[0] + cw[1] * h[1] + cw[2] * h[2]
                     + cpo * po + cpa * pa + cpb * pb + bcv)
                outb[pl.ds(o, 16)] = _sc_tanh(_sc_tanh(z))
                return 0

            lax.fori_loop(0, CBLK // 16, vec_fn, 0)
            pltpu.sync_copy(outb, x1out.at[pl.ds(comp * N + base + off, CBLK)])
            return 0

        lax.fori_loop(0, NCBLK, cblk_fn, 0)


def _sc_layer0(xT, adjf, wv):
    return pl.kernel(
        _sc_layer0_body,
        out_type=(
            jax.ShapeDtypeStruct((C * N,), jnp.float32),
            jax.ShapeDtypeStruct((C * N,), jnp.float32),
        ),
        mesh=plsc.VectorSubcoreMesh(core_axis_name="c", subcore_axis_name="s"),
        compiler_params=_SC_PARAMS,
        scratch_types=[
            pltpu.VMEM((N,), jnp.float32),
            pltpu.VMEM((BLK_A * DEG,), jnp.int32),
            pltpu.VMEM((BLK_A * DEG,), jnp.int32),
            pltpu.VMEM((CHUNK_A,), jnp.float32),
            pltpu.VMEM((CBLK,), jnp.float32),
            pltpu.VMEM((CBLK,), jnp.float32),
            pltpu.VMEM((CBLK,), jnp.float32),
            pltpu.VMEM((CBLK,), jnp.float32),
            pltpu.VMEM((CBLK,), jnp.float32),
            pltpu.VMEM((40,), jnp.float32),
            pltpu.SemaphoreType.DMA,
            pltpu.SemaphoreType.DMA,
        ],
    )(xT, adjf, wv)


_WINDOWS = ((0, 0), (16, 0), (32, 0), (48, 0), (56, 8))


def _sc_pool_sparse_body(x1T, adj2, s3, pooled2, x1s3, plane, s3c, idx0, idx1,
                         row0, row1, poolb, valb, dsem0, dsem1):
    cid = lax.axis_index("c")
    sid = lax.axis_index("s")
    wid = sid * NC + cid

    @pl.when(wid < 15)
    def _():
        comp = wid // 5
        chunk = wid % 5
        pltpu.sync_copy(x1T.at[pl.ds(comp * N, N)], plane)
        lane15 = lax.iota(jnp.int32, 16) == 15
        perms = _lane_perms()
        pltpu.sync_copy(s3.at[pl.ds(chunk * GRP_PER_TEC * GP,
                                    GRP_PER_TEC * GP)], s3c)

        def fire(gi, idxb, rowb, dsem):
            gbase = gi * GP
            for off, _unused in _WINDOWS:
                wvec = s3c[pl.ds(gbase + off, 16)]
                idxb[pl.ds(off, 16)] = jnp.right_shift(wvec, 1)
            pltpu.async_copy(adj2.at[idxb], rowb, dsem)

        bufs = ((idx0, row0, dsem0), (idx1, row1, dsem1))
        fire(0, *bufs[0])

        def round_fn(kb, _):
            for b in (0, 1):
                gi = kb * 2 + b
                idxb, rowb, dsem = bufs[b]

                @pl.when(gi < GRP_PER_TEC)
                def _():
                    @pl.when(gi + 1 < GRP_PER_TEC)
                    def _():
                        fire(gi + 1, *bufs[1 - b])

                    pltpu.make_async_copy(adj2.at[idxb], rowb, dsem).wait()
                    gbase = gi * GP
                    for off, _unused in _WINDOWS:
                        valb[pl.ds(off, 16)] = plsc.load_gather(
                            plane, [s3c[pl.ds(gbase + off, 16)]])
                    for off, l0 in _WINDOWS:
                        pv = jnp.bitwise_and(s3c[pl.ds(gbase + off, 16)], 1)
                        for l in range(l0, 16):
                            p = off + l
                            fb = pv[l] * DEG
                            gs = [plsc.load_gather(
                                plane, [rowb[p, pl.ds(fb + 16 * u, 16)]])
                                for u in range(4)]
                            sv = _tree_total(
                                (gs[0] + gs[1]) + (gs[2] + gs[3]), perms)
                            plsc.store_scatter(
                                poolb, [jnp.full((16,), p, jnp.int32)],
                                sv, mask=lane15)
                    out0 = (comp * NG + chunk * GRP_PER_TEC + gi) * GP
                    pltpu.sync_copy(poolb, pooled2.at[pl.ds(out0, GP)])
                    pltpu.sync_copy(valb, x1s3.at[pl.ds(out0, GP)])
            return 0

        lax.fori_loop(0, (GRP_PER_TEC + 1) // 2, round_fn, 0)


def _sc_pool_sparse(x1T, adj2, s3):
    return pl.kernel(
        _sc_pool_sparse_body,
        out_type=(
            jax.ShapeDtypeStruct((C * NG * GP,), jnp.float32),
            jax.ShapeDtypeStruct((C * NG * GP,), jnp.float32),
        ),
        mesh=plsc.VectorSubcoreMesh(core_axis_name="c", subcore_axis_name="s"),
        compiler_params=_SC_PARAMS,
        scratch_types=[
            pltpu.VMEM((N,), jnp.float32),
            pltpu.VMEM((GRP_PER_TEC * GP,), jnp.int32),
            pltpu.VMEM((GP,), jnp.int32),
            pltpu.VMEM((GP,), jnp.int32),
            pltpu.VMEM((GP, 2 * DEG), jnp.int32),
            pltpu.VMEM((GP, 2 * DEG), jnp.int32),
            pltpu.VMEM((GP,), jnp.float32),
            pltpu.VMEM((GP,), jnp.float32),
            pltpu.SemaphoreType.DMA,
            pltpu.SemaphoreType.DMA,
        ],
    )(x1T, adj2, s3)


def _combine_rows(xrows, prows, ws, bs, wc, bc):
    h = [jnp.tanh(ws[c, 0] * xrows[0] + ws[c, 1] * xrows[1]
                  + ws[c, 2] * xrows[2] + bs[c]) for c in range(C)]
    out = []
    for c in range(C):
        acc = bc[c]
        for k in range(C):
            acc = acc + wc[c, k, 0] * h[k] + wc[c, k, 1] * prows[k]
        out.append(jnp.tanh(jnp.tanh(acc)))
    return out


def _tc_finish_body(ws1_ref, bs1_ref, wc1_ref, bc1_ref,
                    ws2_ref, bs2_ref, wc2_ref, bc2_ref,
                    x1_ref, p2_ref, out_ref):
    x1 = [x1_ref[c] for c in range(C)]
    p2 = [p2_ref[c] * (1.0 / DEG) for c in range(C)]
    x2 = _combine_rows(x1, p2, ws1_ref, bs1_ref, wc1_ref, bc1_ref)
    selfs = [x2[c][0:1, :] for c in range(C)]
    pool3 = [jnp.sum(x2[c][1:NG, :], axis=0, keepdims=True) * (1.0 / DEG)
             for c in range(C)]
    out = _combine_rows(selfs, pool3, ws2_ref, bs2_ref, wc2_ref, bc2_ref)
    out_ref[...] = jnp.concatenate(out, axis=0)


def _tc_finish(x1s3, p2, ws1, bs1, wc1, bc1, ws2, bs2, wc2, bc2):
    smem = pl.BlockSpec(memory_space=pltpu.SMEM)
    return pl.pallas_call(
        _tc_finish_body,
        out_shape=jax.ShapeDtypeStruct((C, GP), jnp.float32),
        in_specs=[smem] * 8 + [pl.BlockSpec(), pl.BlockSpec()],
    )(ws1, bs1, wc1, bc1, ws2, bs2, wc2, bc2, x1s3, p2)


@jax.jit
def kernel(x, adj_mat,
           W_self_0, b_self_0, W_comb_0, b_comb_0,
           W_self_1, b_self_1, W_comb_1, b_comb_1,
           W_self_2, b_self_2, W_comb_2, b_comb_2):
    xTf = x.T.reshape(C * N)
    adjf = adj_mat.reshape(N * DEG)
    w0 = jnp.concatenate([W_self_0.ravel(), b_self_0, W_comb_0.ravel(),
                          b_comb_0, jnp.zeros((7,), jnp.float32)])

    last = jnp.arange(N - V, N, dtype=jnp.int32)
    s3 = jnp.zeros((NG, GP), jnp.int32)
    s3 = s3.at[0, :V].set(last)
    s3 = s3.at[1:, :V].set(adj_mat[N - V:, :].T)

    x1f, _pooled_unused = _sc_layer0(xTf, adjf, w0)
    p2, x1s3 = _sc_pool_sparse(x1f, adjf.reshape(N // 2, 2 * DEG),
                               s3.reshape(-1))
    out = _tc_finish(x1s3.reshape(C, NG, GP), p2.reshape(C, NG, GP),
                     W_self_1, b_self_1, W_comb_1, b_comb_1,
                     W_self_2, b_self_2, W_comb_2, b_comb_2)
    return out[:, :V].T[:, :, None]


# speedup vs baseline: 1.3044x; 1.3044x over previous
import functools

import jax
import jax.numpy as jnp
from jax import lax
from jax.experimental import pallas as pl
from jax.experimental.pallas import tpu as pltpu
from jax.experimental.pallas import tpu_sc as plsc

N = 100000
DEG = 64
C = 3
V = 68
NG = DEG + 1
GP = 72

NC = 2
NS = 16

CHUNK_A = N // 10
BLK_A = 80
NBLK_A = CHUNK_A // BLK_A
GRP_PER_TEC = 13

_SC_PARAMS = pltpu.CompilerParams(needs_layout_passes=False)


def _node_gather_vec(plane, adjfb, fb):
    gs = [plsc.load_gather(plane, [adjfb[pl.ds(fb + 16 * u, 16)]])
          for u in range(4)]
    return (gs[0] + gs[1]) + (gs[2] + gs[3])


def _node_neighbor_cumsum(plane, adjfb, fb):
    return plsc.cumsum(_node_gather_vec(plane, adjfb, fb))


CBLK = 2000
NCBLK = CHUNK_A // CBLK


def _sc_tanh(z):
    return 1.0 - 2.0 / (jnp.exp(z + z) + 1.0)


def _sel3(t, a0, a1, a2):
    return jnp.where(t == 0, a0, jnp.where(t == 1, a1, a2))


def _sc_layer0_body(xT, adjf, wv, x1out, pooledout, plane, adjb0, adjb1,
                    poolchunk, xk1b, xk2b, pk1b, pk2b, outb, wvb,
                    isem0, isem1):
    cid = lax.axis_index("c")
    sid = lax.axis_index("s")
    wid = sid * NC + cid
    comp = wid // 10
    chunk = wid % 10
    active = wid < 30

    @pl.when(active)
    def _():
        pltpu.sync_copy(xT.at[pl.ds(comp * N, N)], plane)
        pltpu.sync_copy(wv, wvb)
        iota16 = lax.iota(jnp.int32, 16)
        lane15 = iota16 == 15
        lane7 = iota16 == 7
        lanelt8 = iota16 < 8
        perm8 = jnp.bitwise_and(iota16 + 8, 15)
        splat7 = jnp.full((16,), 7, jnp.int32)
        base = chunk * CHUNK_A

        def in_slice(blk):
            return adjf.at[pl.ds((base + blk * BLK_A) * DEG, BLK_A * DEG)]

        bufs = ((adjb0, isem0), (adjb1, isem1))
        pltpu.async_copy(in_slice(0), adjb0, isem0)

        def round_fn(kb, _):
            for b, (adjb, isem) in enumerate(bufs):
                blk = kb * 2 + b

                @pl.when(blk < NBLK_A)
                def _():
                    @pl.when(blk + 1 < NBLK_A)
                    def _():
                        nxt = bufs[1 - b]
                        pltpu.async_copy(in_slice(blk + 1), nxt[0], nxt[1])

                    pltpu.make_async_copy(in_slice(blk), adjb, isem).wait()

                    def pair_fn(vp, _):
                        va = vp * 2
                        ga = _node_gather_vec(plane, adjb, va * DEG)
                        gb = _node_gather_vec(plane, adjb, (va + 1) * DEG)
                        fa = ga + ga.at[perm8].get(mode="promise_in_bounds")
                        fb = gb + gb.at[perm8].get(mode="promise_in_bounds")
                        cs = plsc.cumsum(jnp.where(lanelt8, fa, fb))
                        bv = cs - cs.at[splat7].get(mode="promise_in_bounds")
                        na = blk * BLK_A + va
                        plsc.store_scatter(
                            poolchunk, [jnp.full((16,), na, jnp.int32)],
                            cs, mask=lane7)
                        plsc.store_scatter(
                            poolchunk, [jnp.full((16,), na + 1, jnp.int32)],
                            bv, mask=lane15)
                        return 0

                    lax.fori_loop(0, BLK_A // 2, pair_fn, 0, unroll=8)
            return 0

        lax.fori_loop(0, (NBLK_A + 1) // 2, round_fn, 0)
        pltpu.sync_copy(poolchunk, pooledout.at[pl.ds(comp * N + base,
                                                      CHUNK_A)])

    plsc.subcore_barrier()

    @pl.when(active)
    def _():
        k1 = lax.rem(comp + 1, 3)
        k2 = lax.rem(comp + 2, 3)
        base = chunk * CHUNK_A

        wA = wvb[pl.ds(0, 16)]
        wB = wvb[pl.ds(16, 16)]
        wC = wvb[pl.ds(24, 16)]

        def ext(i):
            if i < 16:
                return wA[i]
            if i < 32:
                return wB[i - 16]
            return wC[i - 24]

        def ws_(c, k):
            return ext(3 * c + k)

        def wc_(c, k, m):
            return ext(12 + 6 * c + 2 * k + m)

        hw = [[_sel3(comp, ws_(c, 0), ws_(c, 1), ws_(c, 2)),
               _sel3(k1, ws_(c, 0), ws_(c, 1), ws_(c, 2)),
               _sel3(k2, ws_(c, 0), ws_(c, 1), ws_(c, 2))]
              for c in range(3)]
        bsv = [ext(9 + c) for c in range(3)]
        cw = [_sel3(comp, wc_(0, k, 0), wc_(1, k, 0), wc_(2, k, 0))
              for k in range(3)]
        cpo = _sel3(comp, wc_(0, 0, 1), wc_(1, 1, 1), wc_(2, 2, 1)) * (1.0 / DEG)
        cpa = _sel3(comp, wc_(0, 1, 1), wc_(1, 2, 1), wc_(2, 0, 1)) * (1.0 / DEG)
        cpb = _sel3(comp, wc_(0, 2, 1), wc_(1, 0, 1), wc_(2, 1, 1)) * (1.0 / DEG)
        bcv = _sel3(comp, ext(30), ext(31), ext(32))

        def cblk_fn(cb, _):
            off = cb * CBLK
            cps = [
                pltpu.async_copy(xT.at[pl.ds(k1 * N + base + off, CBLK)],
                                 xk1b, isem0),
                pltpu.async_copy(xT.at[pl.ds(k2 * N + base + off, CBLK)],
                                 xk2b, isem0),
                pltpu.async_copy(
                    pooledout.at[pl.ds(k1 * N + base + off, CBLK)],
                    pk1b, isem0),
                pltpu.async_copy(
                    pooledout.at[pl.ds(k2 * N + base + off, CBLK)],
                    pk2b, isem0),
            ]
            for cp in cps:
                cp.wait()

            def vec_fn(v, _):
                o = v * 16
                xo = plane[pl.ds(base + off + o, 16)]
                xa = xk1b[pl.ds(o, 16)]
                xb = xk2b[pl.ds(o, 16)]
                po = poolchunk[pl.ds(off + o, 16)]
                pa = pk1b[pl.ds(o, 16)]
                pb = pk2b[pl.ds(o, 16)]
                h = [_sc_tanh(hw[c][0] * xo + hw[c][1] * xa + hw[c][2] * xb
                              + bsv[c]) for c in range(3)]
                z = (cw[0] * h[0] + cw[1] * h[1] + cw[2] * h[2]
                     + cpo * po + cpa * pa + cpb * pb + bcv)
                outb[pl.ds(o, 16)] = _sc_tanh(_sc_tanh(z))
                return 0

            lax.fori_loop(0, CBLK // 16, vec_fn, 0)
            pltpu.sync_copy(outb, x1out.at[pl.ds(comp * N + base + off, CBLK)])
            return 0

        lax.fori_loop(0, NCBLK, cblk_fn, 0)


def _sc_layer0(xT, adjf, wv):
    return pl.kernel(
        _sc_layer0_body,
        out_type=(
            jax.ShapeDtypeStruct((C * N,), jnp.float32),
            jax.ShapeDtypeStruct((C * N,), jnp.float32),
        ),
        mesh=plsc.VectorSubcoreMesh(core_axis_name="c", subcore_axis_name="s"),
        compiler_params=_SC_PARAMS,
        scratch_types=[
            pltpu.VMEM((N,), jnp.float32),
            pltpu.VMEM((BLK_A * DEG,), jnp.int32),
            pltpu.VMEM((BLK_A * DEG,), jnp.int32),
            pltpu.VMEM((CHUNK_A,), jnp.float32),
            pltpu.VMEM((CBLK,), jnp.float32),
            pltpu.VMEM((CBLK,), jnp.float32),
            pltpu.VMEM((CBLK,), jnp.float32),
            pltpu.VMEM((CBLK,), jnp.float32),
            pltpu.VMEM((CBLK,), jnp.float32),
            pltpu.VMEM((40,), jnp.float32),
            pltpu.SemaphoreType.DMA,
            pltpu.SemaphoreType.DMA,
        ],
    )(xT, adjf, wv)


_WINDOWS = ((0, 0), (16, 0), (32, 0), (48, 0), (56, 8))


def _sc_pool_sparse_body(x1T, adj2, s3, pooled2, x1s3, plane, s3c, idx0, idx1,
                         row0, row1, poolb, valb, dsem0, dsem1):
    cid = lax.axis_index("c")
    sid = lax.axis_index("s")
    wid = sid * NC + cid

    @pl.when(wid < 15)
    def _():
        comp = wid // 5
        chunk = wid % 5
        pltpu.sync_copy(x1T.at[pl.ds(comp * N, N)], plane)
        lane15 = lax.iota(jnp.int32, 16) == 15
        pltpu.sync_copy(s3.at[pl.ds(chunk * GRP_PER_TEC * GP,
                                    GRP_PER_TEC * GP)], s3c)

        def fire(gi, idxb, rowb, dsem):
            gbase = gi * GP
            for off, _unused in _WINDOWS:
                wvec = s3c[pl.ds(gbase + off, 16)]
                idxb[pl.ds(off, 16)] = jnp.right_shift(wvec, 1)
            pltpu.async_copy(adj2.at[idxb], rowb, dsem)

        bufs = ((idx0, row0, dsem0), (idx1, row1, dsem1))
        fire(0, *bufs[0])

        def round_fn(kb, _):
            for b in (0, 1):
                gi = kb * 2 + b
                idxb, rowb, dsem = bufs[b]

                @pl.when(gi < GRP_PER_TEC)
                def _():
                    @pl.when(gi + 1 < GRP_PER_TEC)
                    def _():
                        fire(gi + 1, *bufs[1 - b])

                    pltpu.make_async_copy(adj2.at[idxb], rowb, dsem).wait()
                    gbase = gi * GP
                    for off, _unused in _WINDOWS:
                        valb[pl.ds(off, 16)] = plsc.load_gather(
                            plane, [s3c[pl.ds(gbase + off, 16)]])
                    for off, l0 in _WINDOWS:
                        pv = jnp.bitwise_and(s3c[pl.ds(gbase + off, 16)], 1)
                        for l in range(l0, 16):
                            p = off + l
                            fb = pv[l] * DEG
                            gs = [plsc.load_gather(
                                plane, [rowb[p, pl.ds(fb + 16 * u, 16)]])
                                for u in range(4)]
                            sv = plsc.cumsum((gs[0] + gs[1]) + (gs[2] + gs[3]))
                            plsc.store_scatter(
                                poolb, [jnp.full((16,), p, jnp.int32)],
                                sv, mask=lane15)
                    out0 = (comp * NG + chunk * GRP_PER_TEC + gi) * GP
                    pltpu.sync_copy(poolb, pooled2.at[pl.ds(out0, GP)])
                    pltpu.sync_copy(valb, x1s3.at[pl.ds(out0, GP)])
            return 0

        lax.fori_loop(0, (GRP_PER_TEC + 1) // 2, round_fn, 0)


def _sc_pool_sparse(x1T, adj2, s3):
    return pl.kernel(
        _sc_pool_sparse_body,
        out_type=(
            jax.ShapeDtypeStruct((C * NG * GP,), jnp.float32),
            jax.ShapeDtypeStruct((C * NG * GP,), jnp.float32),
        ),
        mesh=plsc.VectorSubcoreMesh(core_axis_name="c", subcore_axis_name="s"),
        compiler_params=_SC_PARAMS,
        scratch_types=[
            pltpu.VMEM((N,), jnp.float32),
            pltpu.VMEM((GRP_PER_TEC * GP,), jnp.int32),
            pltpu.VMEM((GP,), jnp.int32),
            pltpu.VMEM((GP,), jnp.int32),
            pltpu.VMEM((GP, 2 * DEG), jnp.int32),
            pltpu.VMEM((GP, 2 * DEG), jnp.int32),
            pltpu.VMEM((GP,), jnp.float32),
            pltpu.VMEM((GP,), jnp.float32),
            pltpu.SemaphoreType.DMA,
            pltpu.SemaphoreType.DMA,
        ],
    )(x1T, adj2, s3)


def _combine_rows(xrows, prows, ws, bs, wc, bc):
    h = [jnp.tanh(ws[c, 0] * xrows[0] + ws[c, 1] * xrows[1]
                  + ws[c, 2] * xrows[2] + bs[c]) for c in range(C)]
    out = []
    for c in range(C):
        acc = bc[c]
        for k in range(C):
            acc = acc + wc[c, k, 0] * h[k] + wc[c, k, 1] * prows[k]
        out.append(jnp.tanh(jnp.tanh(acc)))
    return out


def _tc_finish_body(ws1_ref, bs1_ref, wc1_ref, bc1_ref,
                    ws2_ref, bs2_ref, wc2_ref, bc2_ref,
                    x1_ref, p2_ref, out_ref):
    x1 = [x1_ref[c] for c in range(C)]
    p2 = [p2_ref[c] * (1.0 / DEG) for c in range(C)]
    x2 = _combine_rows(x1, p2, ws1_ref, bs1_ref, wc1_ref, bc1_ref)
    selfs = [x2[c][0:1, :] for c in range(C)]
    pool3 = [jnp.sum(x2[c][1:NG, :], axis=0, keepdims=True) * (1.0 / DEG)
             for c in range(C)]
    out = _combine_rows(selfs, pool3, ws2_ref, bs2_ref, wc2_ref, bc2_ref)
    out_ref[...] = jnp.concatenate(out, axis=0)


def _tc_finish(x1s3, p2, ws1, bs1, wc1, bc1, ws2, bs2, wc2, bc2):
    smem = pl.BlockSpec(memory_space=pltpu.SMEM)
    return pl.pallas_call(
        _tc_finish_body,
        out_shape=jax.ShapeDtypeStruct((C, GP), jnp.float32),
        in_specs=[smem] * 8 + [pl.BlockSpec(), pl.BlockSpec()],
    )(ws1, bs1, wc1, bc1, ws2, bs2, wc2, bc2, x1s3, p2)


@jax.jit
def kernel(x, adj_mat,
           W_self_0, b_self_0, W_comb_0, b_comb_0,
           W_self_1, b_self_1, W_comb_1, b_comb_1,
           W_self_2, b_self_2, W_comb_2, b_comb_2):
    xTf = x.T.reshape(C * N)
    adjf = adj_mat.reshape(N * DEG)
    w0 = jnp.concatenate([W_self_0.ravel(), b_self_0, W_comb_0.ravel(),
                          b_comb_0, jnp.zeros((7,), jnp.float32)])

    last = jnp.arange(N - V, N, dtype=jnp.int32)
    s3 = jnp.zeros((NG, GP), jnp.int32)
    s3 = s3.at[0, :V].set(last)
    s3 = s3.at[1:, :V].set(adj_mat[N - V:, :].T)

    x1f, _pooled_unused = _sc_layer0(xTf, adjf, w0)
    p2, x1s3 = _sc_pool_sparse(x1f, adjf.reshape(N // 2, 2 * DEG),
                               s3.reshape(-1))
    out = _tc_finish(x1s3.reshape(C, NG, GP), p2.reshape(C, NG, GP),
                     W_self_1, b_self_1, W_comb_1, b_comb_1,
                     W_self_2, b_self_2, W_comb_2, b_comb_2)
    return out[:, :V].T[:, :, None]
